# baseline (speedup 1.0000x reference)
import math

import jax
import jax.numpy as jnp
from jax import lax
from jax.experimental import pallas as pl
from jax.experimental.pallas import tpu as pltpu
from jax.experimental.pallas import tpu_sc as plsc

NC, NS, LANES = 2, 16, 16
NW = NC * NS
G = 2000
CI = 80
RPB = G // CI

_F32 = jnp.float32
_I32 = jnp.int32

_COS05 = float(math.cos(0.5))
_COS15 = float(math.cos(1.5))
_COS20 = float(math.cos(2.0))
_PI = float(math.pi)


def _rsqrt(x):
    i = plsc.bitcast(x, _I32)
    y = plsc.bitcast(jnp.asarray(0x5F3759DF, _I32) - (i >> 1), _F32)
    y = y * (1.5 - 0.5 * x * y * y)
    y = y * (1.5 - 0.5 * x * y * y)
    y = y * (1.5 - 0.5 * x * y * y)
    return y


def _sqrt(x):
    xc = jnp.maximum(x, 1e-30)
    return xc * _rsqrt(xc)


def _arccos(x):
    ax = jnp.abs(x)
    s = _sqrt(jnp.maximum(1.0 - ax, 0.0))
    p = -0.0012624911
    p = p * ax + 0.0066700901
    p = p * ax - 0.0170881256
    p = p * ax + 0.0308918810
    p = p * ax - 0.0501743046
    p = p * ax + 0.0889789874
    p = p * ax - 0.2145988016
    p = p * ax + 1.5707963050
    p = s * p
    return jnp.where(x >= 0.0, p, _PI - p)


def _log_1_3(v):
    u = (v - 1.0) / (v + 1.0)
    t = u * u
    s = 1.0 / 13.0
    s = s * t + 1.0 / 11.0
    s = s * t + 1.0 / 9.0
    s = s * t + 1.0 / 7.0
    s = s * t + 1.0 / 5.0
    s = s * t + 1.0 / 3.0
    s = s * t + 1.0
    return 2.0 * u * s


def _cols16(rows_ref, i, ncols=4):
    ln = lax.iota(_I32, 16) + i * 16
    return [plsc.load_gather(rows_ref, [ln, jnp.full((16,), c, _I32)])
            for c in range(ncols)]


def _sc_loss_kernel(N, E, A, T, P, Np):
    slice_n = Np // NS

    def body(table_h, src_h, dst_h, pvp_h, pbl_h, bt_h,
             ai_h, aj_h, ak_h, pa_h,
             ti_h, tj_h, tk_h, tl_h, ps_h, pc_h, b0_h, b1_h, b2_h,
             ns_h, nd_h, pr_h,
             sums_out, pv_out,
             i0, i1, i2, i3, vals_b,
             r0, r1, r2, r3,
             p0, p1, p2, p3, p4, btb,
             pv_sh, zero_v, acc_v, sem_l, sem_g, sem_s):
        cid = lax.axis_index("c")
        sid = lax.axis_index("s")
        wid = cid * NS + sid

        def zbody(k, c):
            zero_v[pl.ds(k * 16, 16)] = jnp.zeros((16,), _F32)
            return c
        lax.fori_loop(0, slice_n // 16, zbody, 0)
        for ph in range(5):
            acc_v[ph] = jnp.zeros((16,), _F32)
        pltpu.sync_copy(zero_v, pv_sh.at[pl.ds(sid * slice_n, slice_n)])
        plsc.subcore_barrier()

        def lin_copy(src, dst):
            pltpu.sync_copy(src, dst)
            return lambda: None

        def run_phase(nfull, idx_refs, idx_bufs, row_bufs, pred_refs,
                      pred_bufs, group_fn, acc_ids, scatter=False):
            nt = -(-nfull // NW)

            def t_body(t, c):
                j = t * NW + lax.rem(wid + t, NW)

                @pl.when(j < nfull)
                def _():
                    waits = []
                    for src, dstb in zip(idx_refs, idx_bufs):
                        waits.append(lin_copy(src.at[pl.ds(j * RPB, RPB)], dstb))
                    for src, dstb in zip(pred_refs, pred_bufs):
                        waits.append(lin_copy(src.at[pl.ds(j * G, G)], dstb))
                    if scatter:
                        waits.append(
                            lin_copy(pvp_h.at[pl.ds(j * RPB, RPB)], vals_b))
                    for w in waits:
                        w()

                    def issue(r, c2):
                        for ib, rb in zip(idx_bufs, row_bufs):
                            pltpu.sync_copy(table_h.at[ib.at[r]],
                                            rb.at[pl.ds(r * CI, CI)])
                        if scatter:
                            pltpu.sync_copy(vals_b.at[r], pv_sh.at[i0.at[r]],
                                            add=True)
                            pltpu.sync_copy(vals_b.at[r], pv_sh.at[i1.at[r]],
                                            add=True)
                        return c2
                    lax.fori_loop(0, RPB, issue, 0)

                    init = tuple(acc_v[a] for a in acc_ids)
                    fin = lax.fori_loop(0, G // 16, group_fn, init)
                    for a, v in zip(acc_ids, fin):
                        acc_v[a] = v
                return c
            lax.fori_loop(0, nt, t_body, 0)

        def bond_group(i, accs):
            (acc,) = accs
            xs, ys, zs, hs = _cols16(r0, i)
            xd, yd, zd, hd = _cols16(r1, i)
            dx, dy, dz = xs - xd, ys - yd, zs - zd
            tl = _sqrt(dx * dx + dy * dy + dz * dz)
            pb = p0[pl.ds(i * 16, 16)]
            e = pb - tl
            e = e * e
            bt = btb[pl.ds(i * 16, 16)]
            w = jnp.where(bt == 2, 2.0, 1.0)
            w = jnp.where(bt == 3, 3.0, w)
            w = jnp.where(bt == 12, 1.5, w)
            w = jnp.where(hs + hd > 0.0, w * 0.3, w)
            return (acc + w * e,)

        run_phase(E // G, [src_h, dst_h], [i0, i1], [r0, r1],
                  [pbl_h, bt_h], [p0, btb], bond_group, (0,), scatter=True)

        def angle_group(i, accs):
            (acc,) = accs
            xi, yi, zi, hi = _cols16(r0, i)
            xj, yj, zj, hj = _cols16(r1, i)
            xk, yk, zk, hk = _cols16(r2, i)
            v1x, v1y, v1z = xi - xj, yi - yj, zi - zj
            v2x, v2y, v2z = xk - xj, yk - yj, zk - zj
            dot = v1x * v2x + v1y * v2y + v1z * v2z
            n1 = v1x * v1x + v1y * v1y + v1z * v1z
            n2 = v2x * v2x + v2y * v2y + v2z * v2z
            den = jnp.maximum(_sqrt(n1) * _sqrt(n2), 1e-8)
            cv = jnp.minimum(jnp.maximum(dot / den, -1.0), 1.0)
            ta = _arccos(cv)
            d = p0[pl.ds(i * 16, 16)] - ta
            wv = jnp.where(hi + hj + hk > 0.0, 0.3, 1.0)
            return (acc + wv * d * d,)

        run_phase(A // G, [ai_h, aj_h, ak_h], [i0, i1, i2], [r0, r1, r2],
                  [pa_h], [p0], angle_group, (1,))

        def torsion_group(i, accs):
            acc_t, acc_b = accs
            xi, yi, zi, hi = _cols16(r0, i)
            xj, yj, zj, hj = _cols16(r1, i)
            xk, yk, zk, hk = _cols16(r2, i)
            xl, yl, zl, hl = _cols16(r3, i)
            u1x, u1y, u1z = xj - xi, yj - yi, zj - zi
            u2x, u2y, u2z = xk - xj, yk - yj, zk - zj
            u3x, u3y, u3z = xl - xk, yl - yk, zl - zk
            c1x = u1y * u2z - u1z * u2y + 1e-6
            c1y = u1z * u2x - u1x * u2z + 1e-6
            c1z = u1x * u2y - u1y * u2x + 1e-6
            c2x = u2y * u3z - u2z * u3y + 1e-6
            c2y = u2z * u3x - u2x * u3z + 1e-6
            c2z = u2x * u3y - u2y * u3x + 1e-6
            ex, ey, ez = u2x + 1e-6, u2y + 1e-6, u2z + 1e-6
            q1 = 1.0 / jnp.maximum(
                _sqrt(c1x * c1x + c1y * c1y + c1z * c1z), 1e-12)
            q2 = 1.0 / jnp.maximum(
                _sqrt(c2x * c2x + c2y * c2y + c2z * c2z), 1e-12)
            qe = 1.0 / jnp.maximum(_sqrt(ex * ex + ey * ey + ez * ez), 1e-12)
            n1x, n1y, n1z = c1x * q1, c1y * q1, c1z * q1
            n2x, n2y, n2z = c2x * q2, c2y * q2, c2z * q2
            ux, uy, uz = ex * qe, ey * qe, ez * qe
            cosv = n1x * n2x + n1y * n2y + n1z * n2z
            crx = n1y * n2z - n1z * n2y
            cry = n1z * n2x - n1x * n2z
            crz = n1x * n2y - n1y * n2x
            sinv = crx * ux + cry * uy + crz * uz
            sl = pl.ds(i * 16, 16)
            es = p0[sl] - sinv
            ec = p1[sl] - cosv
            wv = jnp.where(hi + hj + hk + hl > 0.0, 0.3, 1.0)
            acc_t = acc_t + wv * (es * es + ec * ec)
            b0 = p2[sl]
            b1 = p3[sl]
            b2 = p4[sl]
            r = _sqrt(sinv * sinv + cosv * cosv)
            anti = cosv < r * _COS20
            inband = (cosv < r * _COS05) & (cosv > r * _COS15)
            gp = (sinv > 0.0) & inband
            gm = (sinv < 0.0) & inband
            act = anti | gp | gm
            mx = jnp.maximum(b0, jnp.maximum(b1, b2))
            se = jnp.exp(b0 - mx) + jnp.exp(b1 - mx) + jnp.exp(b2 - mx)
            lse = _log_1_3(se)
            xsel = jnp.where(anti, b0, jnp.where(gp, b1, b2))
            acc_b = acc_b + jnp.where(act, mx + lse - xsel, 0.0)
            return acc_t, acc_b

        run_phase(T // G, [ti_h, tj_h, tk_h, tl_h], [i0, i1, i2, i3],
                  [r0, r1, r2, r3], [ps_h, pc_h, b0_h, b1_h, b2_h],
                  [p0, p1, p2, p3, p4], torsion_group, (2, 3))

        def rep_group(i, accs):
            (acc,) = accs
            xs, ys, zs, _hs = _cols16(r0, i)
            xd, yd, zd, _hd = _cols16(r1, i)
            dx, dy, dz = xs - xd, ys - yd, zs - zd
            d = _sqrt(dx * dx + dy * dy + dz * dz)
            pr = p0[pl.ds(i * 16, 16)]
            return (acc + jnp.maximum(pr - d + 0.3, 0.0),)

        run_phase(P // G, [ns_h, nd_h], [i0, i1], [r0, r1],
                  [pr_h], [p0], rep_group, (4,))

        plsc.subcore_barrier()
        pltpu.sync_copy(pv_sh.at[pl.ds(sid * slice_n, slice_n)],
                        pv_out.at[cid, pl.ds(sid * slice_n, slice_n)])
        pltpu.sync_copy(acc_v, sums_out.at[wid])

    return body


def _tc_combine_kernel(N, E, A, T, P, B, nrow):
    def body(sums_ref, pv_ref, mv_ref, pd_ref, td_ref, out_ref):
        s = sums_ref[...]
        col = lax.broadcasted_iota(_I32, s.shape, 1) // 16

        def msum(ph):
            return jnp.sum(jnp.where(col == ph, s, 0.0))

        bond = msum(0) / E
        angle = msum(1) / A
        torsion = msum(2) / (2.0 * T)
        tbin = msum(3) / T
        rep = msum(4) / P
        pv = (pv_ref[0:nrow, :] + pv_ref[nrow:2 * nrow, :]) * 0.5
        mv = mv_ref[...].astype(_F32)
        val = jnp.sum(jnp.maximum(pv - mv, 0.0)) / N
        q = td_ref[...] + 1e-8
        dd = jnp.sum(q * (jnp.log(q) - jnp.log(pd_ref[...] + 1e-8))) / B
        total = (bond + 0.5 * angle + 0.3 * (torsion + tbin)
                 + 0.4 * rep + 0.3 * val + 0.5 * dd)
        lane = lax.broadcasted_iota(_I32, (1, 128), 1)
        out = jnp.where(lane == 0, bond, 0.0)
        out = out + jnp.where(lane == 1, angle, 0.0)
        out = out + jnp.where(lane == 2, torsion, 0.0)
        out = out + jnp.where(lane == 3, tbin, 0.0)
        out = out + jnp.where(lane == 4, rep, 0.0)
        out = out + jnp.where(lane == 5, val, 0.0)
        out = out + jnp.where(lane == 6, dd, 0.0)
        out = out + jnp.where(lane == 8, total, 0.0)
        out_ref[...] = out
    return body


def kernel(pos_true, batch_idx, edge_index, pred_bond_lengths, angle_triplets,
           pred_angles, torsion_quads, pred_torsions, pred_torsion_bins,
           nonbond_pairs, pred_repulsion, pred_valence_probs, bond_types,
           atom_max_valences, atom_is_hydrogen, pred_dist_distribution,
           true_dist_distribution):
    N = pos_true.shape[0]
    E = edge_index.shape[1]
    A = angle_triplets.shape[0]
    T = torsion_quads.shape[0]
    P = nonbond_pairs.shape[1]
    B = pred_dist_distribution.shape[0]
    Np = -(-N // (NS * 128)) * (NS * 128)

    table = jnp.concatenate(
        [pos_true, atom_is_hydrogen.astype(_F32)[:, None],
         jnp.zeros((N, 4), _F32)], axis=1)
    r2 = lambda a: a.reshape(-1, CI)

    sc = pl.kernel(
        _sc_loss_kernel(N, E, A, T, P, Np),
        out_type=[jax.ShapeDtypeStruct((NW, 5, 16), _F32),
                  jax.ShapeDtypeStruct((NC, Np), _F32)],
        mesh=plsc.VectorSubcoreMesh(core_axis_name="c", subcore_axis_name="s"),
        compiler_params=pltpu.CompilerParams(use_tc_tiling_on_sc=False,
                                             needs_layout_passes=False),
        scratch_types=[
            pltpu.VMEM((RPB, CI), _I32), pltpu.VMEM((RPB, CI), _I32),
            pltpu.VMEM((RPB, CI), _I32), pltpu.VMEM((RPB, CI), _I32),
            pltpu.VMEM((RPB, CI), _F32),
            pltpu.VMEM((G, 8), _F32), pltpu.VMEM((G, 8), _F32),
            pltpu.VMEM((G, 8), _F32), pltpu.VMEM((G, 8), _F32),
            pltpu.VMEM((G,), _F32), pltpu.VMEM((G,), _F32),
            pltpu.VMEM((G,), _F32), pltpu.VMEM((G,), _F32),
            pltpu.VMEM((G,), _F32), pltpu.VMEM((G,), _I32),
            pltpu.VMEM_SHARED((Np,), _F32),
            pltpu.VMEM((Np // NS,), _F32),
            pltpu.VMEM((5, 16), _F32),
            pltpu.SemaphoreType.DMA, pltpu.SemaphoreType.DMA,
            pltpu.SemaphoreType.DMA,
        ],
    )
    sums, pv = sc(table, r2(edge_index[0]), r2(edge_index[1]),
                  r2(pred_valence_probs),
                  pred_bond_lengths, bond_types,
                  r2(angle_triplets[:, 0]), r2(angle_triplets[:, 1]),
                  r2(angle_triplets[:, 2]), pred_angles,
                  r2(torsion_quads[:, 0]), r2(torsion_quads[:, 1]),
                  r2(torsion_quads[:, 2]), r2(torsion_quads[:, 3]),
                  pred_torsions[:, 0], pred_torsions[:, 1],
                  pred_torsion_bins[:, 0], pred_torsion_bins[:, 1],
                  pred_torsion_bins[:, 2],
                  r2(nonbond_pairs[0]), r2(nonbond_pairs[1]), pred_repulsion)

    nrow = Np // 128
    mv = jnp.pad(atom_max_valences, (0, Np - N),
                 constant_values=10**9).reshape(nrow, 128)
    out = pl.pallas_call(
        _tc_combine_kernel(N, E, A, T, P, B, nrow),
        out_shape=jax.ShapeDtypeStruct((1, 128), _F32),
    )(sums.reshape(NW, 80), pv.reshape(2 * nrow, 128), mv,
      pred_dist_distribution, true_dist_distribution)
    return out[0, :9]


# speedup vs baseline: 3.5229x; 3.5229x over previous
import math

import jax
import jax.numpy as jnp
from jax import lax
from jax.experimental import pallas as pl
from jax.experimental.pallas import tpu as pltpu
from jax.experimental.pallas import tpu_sc as plsc

NC, NS, LANES = 2, 16, 16
NW = NC * NS
G = 2000
CI = 80
RPB = G // CI

_F32 = jnp.float32
_I32 = jnp.int32

_COS05 = float(math.cos(0.5))
_COS15 = float(math.cos(1.5))
_COS20 = float(math.cos(2.0))
_PI = float(math.pi)


def _rsqrt(x):
    i = plsc.bitcast(x, _I32)
    y = plsc.bitcast(jnp.asarray(0x5F3759DF, _I32) - (i >> 1), _F32)
    y = y * (1.5 - 0.5 * x * y * y)
    y = y * (1.5 - 0.5 * x * y * y)
    y = y * (1.5 - 0.5 * x * y * y)
    return y


def _sqrt(x):
    xc = jnp.maximum(x, 1e-30)
    return xc * _rsqrt(xc)


def _arccos(x):
    ax = jnp.abs(x)
    s = _sqrt(jnp.maximum(1.0 - ax, 0.0))
    p = -0.0012624911
    p = p * ax + 0.0066700901
    p = p * ax - 0.0170881256
    p = p * ax + 0.0308918810
    p = p * ax - 0.0501743046
    p = p * ax + 0.0889789874
    p = p * ax - 0.2145988016
    p = p * ax + 1.5707963050
    p = s * p
    return jnp.where(x >= 0.0, p, _PI - p)


def _log_1_3(v):
    u = (v - 1.0) / (v + 1.0)
    t = u * u
    s = 1.0 / 13.0
    s = s * t + 1.0 / 11.0
    s = s * t + 1.0 / 9.0
    s = s * t + 1.0 / 7.0
    s = s * t + 1.0 / 5.0
    s = s * t + 1.0 / 3.0
    s = s * t + 1.0
    return 2.0 * u * s


def _cols16(rows_ref, i, ncols=4):
    ln = lax.iota(_I32, 16) + i * 16
    return [plsc.load_gather(rows_ref, [ln, jnp.full((16,), c, _I32)])
            for c in range(ncols)]


def _sc_loss_kernel(N, E, A, T, P, Np):
    slice_n = Np // NS

    def body(table_h, src_h, dst_h, pvp_h, pbl_h, bt_h,
             ai_h, aj_h, ak_h, pa_h,
             ti_h, tj_h, tk_h, tl_h, ps_h, pc_h, b0_h, b1_h, b2_h,
             ns_h, nd_h, pr_h,
             sums_out, pv_out,
             i0, i1, i2, i3, vals_b,
             r0, r1, r2, r3,
             p0, p1, p2, p3, p4, btb,
             pv_sh, zero_v, acc_v, sem_l, sem_g, sem_s):
        cid = lax.axis_index("c")
        sid = lax.axis_index("s")
        wid = cid * NS + sid

        def zbody(k, c):
            zero_v[pl.ds(k * 16, 16)] = jnp.zeros((16,), _F32)
            return c
        lax.fori_loop(0, slice_n // 16, zbody, 0)
        for ph in range(5):
            acc_v[ph] = jnp.zeros((16,), _F32)
        pltpu.sync_copy(zero_v, pv_sh.at[pl.ds(sid * slice_n, slice_n)])
        plsc.subcore_barrier()

        def lin_copy(src, dst):
            pltpu.async_copy(src, dst, sem_l)
            return pltpu.make_async_copy(src, dst, sem_l).wait

        def run_phase(nfull, idx_refs, idx_bufs, row_bufs, pred_refs,
                      pred_bufs, group_fn, acc_ids, scatter=False):
            nt = -(-nfull // NW)

            def t_body(t, c):
                j = t * NW + lax.rem(wid + t, NW)

                @pl.when(j < nfull)
                def _():
                    waits = []
                    for src, dstb in zip(idx_refs, idx_bufs):
                        waits.append(lin_copy(src.at[pl.ds(j * RPB, RPB)], dstb))
                    for src, dstb in zip(pred_refs, pred_bufs):
                        waits.append(lin_copy(src.at[pl.ds(j * G, G)], dstb))
                    if scatter:
                        waits.append(
                            lin_copy(pvp_h.at[pl.ds(j * RPB, RPB)], vals_b))
                    for w in waits:
                        w()

                    def issue(r, c2):
                        for ib, rb in zip(idx_bufs, row_bufs):
                            pltpu.async_copy(table_h.at[ib.at[r]],
                                             rb.at[pl.ds(r * CI, CI)], sem_g)
                        if scatter:
                            pltpu.async_copy(vals_b.at[r], pv_sh.at[i0.at[r]],
                                             sem_s, add=True)
                            pltpu.async_copy(vals_b.at[r], pv_sh.at[i1.at[r]],
                                             sem_s, add=True)
                        return c2
                    lax.fori_loop(0, RPB, issue, 0)

                    def drain(r, c2):
                        for ib, rb in zip(idx_bufs, row_bufs):
                            pltpu.make_async_copy(
                                table_h.at[ib.at[r]],
                                rb.at[pl.ds(r * CI, CI)], sem_g).wait()
                        if scatter:
                            pltpu.make_async_copy(
                                vals_b.at[r], pv_sh.at[i0.at[r]], sem_s).wait()
                            pltpu.make_async_copy(
                                vals_b.at[r], pv_sh.at[i1.at[r]], sem_s).wait()
                        return c2
                    lax.fori_loop(0, RPB, drain, 0)

                    init = tuple(acc_v[a] for a in acc_ids)
                    fin = lax.fori_loop(0, G // 16, group_fn, init)
                    for a, v in zip(acc_ids, fin):
                        acc_v[a] = v
                return c
            lax.fori_loop(0, nt, t_body, 0)

        def bond_group(i, accs):
            (acc,) = accs
            xs, ys, zs, hs = _cols16(r0, i)
            xd, yd, zd, hd = _cols16(r1, i)
            dx, dy, dz = xs - xd, ys - yd, zs - zd
            tl = _sqrt(dx * dx + dy * dy + dz * dz)
            pb = p0[pl.ds(i * 16, 16)]
            e = pb - tl
            e = e * e
            bt = btb[pl.ds(i * 16, 16)]
            w = jnp.where(bt == 2, 2.0, 1.0)
            w = jnp.where(bt == 3, 3.0, w)
            w = jnp.where(bt == 12, 1.5, w)
            w = jnp.where(hs + hd > 0.0, w * 0.3, w)
            return (acc + w * e,)

        run_phase(E // G, [src_h, dst_h], [i0, i1], [r0, r1],
                  [pbl_h, bt_h], [p0, btb], bond_group, (0,), scatter=True)

        def angle_group(i, accs):
            (acc,) = accs
            xi, yi, zi, hi = _cols16(r0, i)
            xj, yj, zj, hj = _cols16(r1, i)
            xk, yk, zk, hk = _cols16(r2, i)
            v1x, v1y, v1z = xi - xj, yi - yj, zi - zj
            v2x, v2y, v2z = xk - xj, yk - yj, zk - zj
            dot = v1x * v2x + v1y * v2y + v1z * v2z
            n1 = v1x * v1x + v1y * v1y + v1z * v1z
            n2 = v2x * v2x + v2y * v2y + v2z * v2z
            den = jnp.maximum(_sqrt(n1) * _sqrt(n2), 1e-8)
            cv = jnp.minimum(jnp.maximum(dot / den, -1.0), 1.0)
            ta = _arccos(cv)
            d = p0[pl.ds(i * 16, 16)] - ta
            wv = jnp.where(hi + hj + hk > 0.0, 0.3, 1.0)
            return (acc + wv * d * d,)

        run_phase(A // G, [ai_h, aj_h, ak_h], [i0, i1, i2], [r0, r1, r2],
                  [pa_h], [p0], angle_group, (1,))

        def torsion_group(i, accs):
            acc_t, acc_b = accs
            xi, yi, zi, hi = _cols16(r0, i)
            xj, yj, zj, hj = _cols16(r1, i)
            xk, yk, zk, hk = _cols16(r2, i)
            xl, yl, zl, hl = _cols16(r3, i)
            u1x, u1y, u1z = xj - xi, yj - yi, zj - zi
            u2x, u2y, u2z = xk - xj, yk - yj, zk - zj
            u3x, u3y, u3z = xl - xk, yl - yk, zl - zk
            c1x = u1y * u2z - u1z * u2y + 1e-6
            c1y = u1z * u2x - u1x * u2z + 1e-6
            c1z = u1x * u2y - u1y * u2x + 1e-6
            c2x = u2y * u3z - u2z * u3y + 1e-6
            c2y = u2z * u3x - u2x * u3z + 1e-6
            c2z = u2x * u3y - u2y * u3x + 1e-6
            ex, ey, ez = u2x + 1e-6, u2y + 1e-6, u2z + 1e-6
            q1 = 1.0 / jnp.maximum(
                _sqrt(c1x * c1x + c1y * c1y + c1z * c1z), 1e-12)
            q2 = 1.0 / jnp.maximum(
                _sqrt(c2x * c2x + c2y * c2y + c2z * c2z), 1e-12)
            qe = 1.0 / jnp.maximum(_sqrt(ex * ex + ey * ey + ez * ez), 1e-12)
            n1x, n1y, n1z = c1x * q1, c1y * q1, c1z * q1
            n2x, n2y, n2z = c2x * q2, c2y * q2, c2z * q2
            ux, uy, uz = ex * qe, ey * qe, ez * qe
            cosv = n1x * n2x + n1y * n2y + n1z * n2z
            crx = n1y * n2z - n1z * n2y
            cry = n1z * n2x - n1x * n2z
            crz = n1x * n2y - n1y * n2x
            sinv = crx * ux + cry * uy + crz * uz
            sl = pl.ds(i * 16, 16)
            es = p0[sl] - sinv
            ec = p1[sl] - cosv
            wv = jnp.where(hi + hj + hk + hl > 0.0, 0.3, 1.0)
            acc_t = acc_t + wv * (es * es + ec * ec)
            b0 = p2[sl]
            b1 = p3[sl]
            b2 = p4[sl]
            r = _sqrt(sinv * sinv + cosv * cosv)
            anti = cosv < r * _COS20
            inband = (cosv < r * _COS05) & (cosv > r * _COS15)
            gp = (sinv > 0.0) & inband
            gm = (sinv < 0.0) & inband
            act = anti | gp | gm
            mx = jnp.maximum(b0, jnp.maximum(b1, b2))
            se = jnp.exp(b0 - mx) + jnp.exp(b1 - mx) + jnp.exp(b2 - mx)
            lse = _log_1_3(se)
            xsel = jnp.where(anti, b0, jnp.where(gp, b1, b2))
            acc_b = acc_b + jnp.where(act, mx + lse - xsel, 0.0)
            return acc_t, acc_b

        run_phase(T // G, [ti_h, tj_h, tk_h, tl_h], [i0, i1, i2, i3],
                  [r0, r1, r2, r3], [ps_h, pc_h, b0_h, b1_h, b2_h],
                  [p0, p1, p2, p3, p4], torsion_group, (2, 3))

        def rep_group(i, accs):
            (acc,) = accs
            xs, ys, zs, _hs = _cols16(r0, i)
            xd, yd, zd, _hd = _cols16(r1, i)
            dx, dy, dz = xs - xd, ys - yd, zs - zd
            d = _sqrt(dx * dx + dy * dy + dz * dz)
            pr = p0[pl.ds(i * 16, 16)]
            return (acc + jnp.maximum(pr - d + 0.3, 0.0),)

        run_phase(P // G, [ns_h, nd_h], [i0, i1], [r0, r1],
                  [pr_h], [p0], rep_group, (4,))

        plsc.subcore_barrier()
        pltpu.sync_copy(pv_sh.at[pl.ds(sid * slice_n, slice_n)],
                        pv_out.at[cid, pl.ds(sid * slice_n, slice_n)])
        pltpu.sync_copy(acc_v, sums_out.at[wid])

    return body


def _tc_combine_kernel(N, E, A, T, P, B, nrow):
    def body(sums_ref, pv_ref, mv_ref, pd_ref, td_ref, out_ref):
        s = sums_ref[...]
        col = lax.broadcasted_iota(_I32, s.shape, 1) // 16

        def msum(ph):
            return jnp.sum(jnp.where(col == ph, s, 0.0))

        bond = msum(0) / E
        angle = msum(1) / A
        torsion = msum(2) / (2.0 * T)
        tbin = msum(3) / T
        rep = msum(4) / P
        pv = (pv_ref[0:nrow, :] + pv_ref[nrow:2 * nrow, :]) * 0.5
        mv = mv_ref[...].astype(_F32)
        val = jnp.sum(jnp.maximum(pv - mv, 0.0)) / N
        q = td_ref[...] + 1e-8
        dd = jnp.sum(q * (jnp.log(q) - jnp.log(pd_ref[...] + 1e-8))) / B
        total = (bond + 0.5 * angle + 0.3 * (torsion + tbin)
                 + 0.4 * rep + 0.3 * val + 0.5 * dd)
        lane = lax.broadcasted_iota(_I32, (1, 128), 1)
        out = jnp.where(lane == 0, bond, 0.0)
        out = out + jnp.where(lane == 1, angle, 0.0)
        out = out + jnp.where(lane == 2, torsion, 0.0)
        out = out + jnp.where(lane == 3, tbin, 0.0)
        out = out + jnp.where(lane == 4, rep, 0.0)
        out = out + jnp.where(lane == 5, val, 0.0)
        out = out + jnp.where(lane == 6, dd, 0.0)
        out = out + jnp.where(lane == 8, total, 0.0)
        out_ref[...] = out
    return body


def kernel(pos_true, batch_idx, edge_index, pred_bond_lengths, angle_triplets,
           pred_angles, torsion_quads, pred_torsions, pred_torsion_bins,
           nonbond_pairs, pred_repulsion, pred_valence_probs, bond_types,
           atom_max_valences, atom_is_hydrogen, pred_dist_distribution,
           true_dist_distribution):
    N = pos_true.shape[0]
    E = edge_index.shape[1]
    A = angle_triplets.shape[0]
    T = torsion_quads.shape[0]
    P = nonbond_pairs.shape[1]
    B = pred_dist_distribution.shape[0]
    Np = -(-N // (NS * 128)) * (NS * 128)

    table = jnp.concatenate(
        [pos_true, atom_is_hydrogen.astype(_F32)[:, None],
         jnp.zeros((N, 4), _F32)], axis=1)
    r2 = lambda a: a.reshape(-1, CI)

    sc = pl.kernel(
        _sc_loss_kernel(N, E, A, T, P, Np),
        out_type=[jax.ShapeDtypeStruct((NW, 5, 16), _F32),
                  jax.ShapeDtypeStruct((NC, Np), _F32)],
        mesh=plsc.VectorSubcoreMesh(core_axis_name="c", subcore_axis_name="s"),
        compiler_params=pltpu.CompilerParams(use_tc_tiling_on_sc=False,
                                             needs_layout_passes=False),
        scratch_types=[
            pltpu.VMEM((RPB, CI), _I32), pltpu.VMEM((RPB, CI), _I32),
            pltpu.VMEM((RPB, CI), _I32), pltpu.VMEM((RPB, CI), _I32),
            pltpu.VMEM((RPB, CI), _F32),
            pltpu.VMEM((G, 8), _F32), pltpu.VMEM((G, 8), _F32),
            pltpu.VMEM((G, 8), _F32), pltpu.VMEM((G, 8), _F32),
            pltpu.VMEM((G,), _F32), pltpu.VMEM((G,), _F32),
            pltpu.VMEM((G,), _F32), pltpu.VMEM((G,), _F32),
            pltpu.VMEM((G,), _F32), pltpu.VMEM((G,), _I32),
            pltpu.VMEM_SHARED((Np,), _F32),
            pltpu.VMEM((Np // NS,), _F32),
            pltpu.VMEM((5, 16), _F32),
            pltpu.SemaphoreType.DMA, pltpu.SemaphoreType.DMA,
            pltpu.SemaphoreType.DMA,
        ],
    )
    sums, pv = sc(table, r2(edge_index[0]), r2(edge_index[1]),
                  r2(pred_valence_probs),
                  pred_bond_lengths, bond_types,
                  r2(angle_triplets[:, 0]), r2(angle_triplets[:, 1]),
                  r2(angle_triplets[:, 2]), pred_angles,
                  r2(torsion_quads[:, 0]), r2(torsion_quads[:, 1]),
                  r2(torsion_quads[:, 2]), r2(torsion_quads[:, 3]),
                  pred_torsions[:, 0], pred_torsions[:, 1],
                  pred_torsion_bins[:, 0], pred_torsion_bins[:, 1],
                  pred_torsion_bins[:, 2],
                  r2(nonbond_pairs[0]), r2(nonbond_pairs[1]), pred_repulsion)

    nrow = Np // 128
    mv = jnp.pad(atom_max_valences, (0, Np - N),
                 constant_values=10**9).reshape(nrow, 128)
    out = pl.pallas_call(
        _tc_combine_kernel(N, E, A, T, P, B, nrow),
        out_shape=jax.ShapeDtypeStruct((1, 128), _F32),
    )(sums.reshape(NW, 80), pv.reshape(2 * nrow, 128), mv,
      pred_dist_distribution, true_dist_distribution)
    return out[0, :9]
